# baseline (speedup 1.0000x reference)
import functools

import jax
import jax.numpy as jnp
from jax import lax
from jax.experimental import pallas as pl
from jax.experimental.pallas import tpu as pltpu
from jax.experimental.pallas import tpu_sc as plsc

_B = 4096
_T = 200
_D = 32
_BLK = 128
_TT = 8
_NCH = _T // _TT


_V = 1000000
_NBLK = _V // 128
_VTAIL = _V - _NBLK * 128


def _make_shuffle(NC: int, NS: int):
    NW = NC * NS
    PER_W = _NBLK // NW + 1

    mesh = plsc.VectorSubcoreMesh(core_axis_name="c", subcore_axis_name="s")

    @functools.partial(
        pl.kernel,
        mesh=mesh,
        compiler_params=pltpu.CompilerParams(use_tc_tiling_on_sc=True,
                                             needs_layout_passes=False),
        out_type=jax.ShapeDtypeStruct((_NBLK + 1, 32, 128), jnp.float32),
        scratch_types=[
            pltpu.VMEM((4, 32, 128), jnp.float32),
            pltpu.SemaphoreType.DMA((4,)),
            pltpu.SemaphoreType.DMA((4,)),
        ],
    )
    def sk(tT_hbm, tbig_hbm, vbuf, s_i, s_o):
        wid = lax.axis_index("s") * NC + lax.axis_index("c")

        def blk_of(i):
            return wid + NW * i

        def in_cp(i, b):
            return pltpu.make_async_copy(
                tT_hbm.at[:, pl.ds(blk_of(i) * 128, 128)], vbuf.at[b],
                s_i.at[b])

        def out_cp(i, b):
            return pltpu.make_async_copy(
                vbuf.at[b], tbig_hbm.at[blk_of(i)], s_o.at[b])

        for j in range(2):
            @pl.when(blk_of(j) < _NBLK)
            def _():
                in_cp(j, j).start()

        def body(i, carry):
            b = i % 4

            @pl.when(jnp.logical_and(i >= 2, blk_of(i - 2) < _NBLK))
            def _():
                out_cp(i - 2, (i - 2) % 4).wait()

            @pl.when(blk_of(i + 2) < _NBLK)
            def _():
                in_cp(i + 2, (i + 2) % 4).start()

            @pl.when(blk_of(i) < _NBLK)
            def _():
                in_cp(i, b).wait()
                out_cp(i, b).start()

            return carry

        lax.fori_loop(0, PER_W, body, 0)
        for i in (PER_W - 2, PER_W - 1):
            @pl.when(blk_of(i) < _NBLK)
            def _():
                out_cp(i, i % 4).wait()

    return sk


def _make_untile(NC: int, NS: int):
    NW = NC * NS
    PER_W = _NBLK // NW + 1

    mesh = plsc.VectorSubcoreMesh(core_axis_name="c", subcore_axis_name="s")

    @functools.partial(
        pl.kernel,
        mesh=mesh,
        compiler_params=pltpu.CompilerParams(use_tc_tiling_on_sc=False,
                                             needs_layout_passes=False),
        out_type=jax.ShapeDtypeStruct((_V // 4, 128), jnp.float32),
        scratch_types=[
            pltpu.VMEM((2, 32, 129), jnp.float32),
            pltpu.VMEM((2, 32, 128), jnp.float32),
            pltpu.SemaphoreType.DMA((2,)),
            pltpu.SemaphoreType.DMA((2,)),
        ],
    )
    def tk(tbig_hbm, ttail_hbm, tlin_hbm, sbuf, obuf, s_i, s_o):
        wid = lax.axis_index("s") * NC + lax.axis_index("c")
        lane = lax.iota(jnp.int32, 16)

        def blk_of(i):
            return wid + NW * i

        def in_cp(i, b):
            return pltpu.make_async_copy(
                tbig_hbm.at[blk_of(i)], sbuf.at[b, :, pl.ds(0, 128)],
                s_i.at[b])

        def out_cp(i, b):
            return pltpu.make_async_copy(
                obuf.at[b], tlin_hbm.at[pl.ds(blk_of(i) * 32, 32), :],
                s_o.at[b])

        def repack(b):
            sb = sbuf.at[b]

            @plsc.parallel_loop(0, 32 * 8, unroll=16)
            def _(kk):
                r = kk >> 3
                h = kk & 7
                rows = lane + (h & 1) * 16
                col = jnp.zeros((16,), jnp.int32) + (4 * r + (h >> 1))
                v = plsc.load_gather(sb, [rows, col])
                obuf[b, r, pl.ds(h * 16, 16)] = v

        @pl.when(blk_of(0) < _NBLK)
        def _():
            in_cp(0, 0).start()

        def body(i, carry):
            b = i % 2
            b1 = (i + 1) % 2

            @pl.when(blk_of(i + 1) < _NBLK)
            def _():
                in_cp(i + 1, b1).start()

            @pl.when(blk_of(i) < _NBLK)
            def _():
                in_cp(i, b).wait()

                @pl.when(i >= 2)
                def _():
                    out_cp(i - 2, b).wait()

                repack(b)
                out_cp(i, b).start()

            return carry

        lax.fori_loop(0, PER_W, body, 0)
        for i in (PER_W - 2, PER_W - 1):
            @pl.when(blk_of(i) < _NBLK)
            def _():
                out_cp(i, i % 2).wait()

        @pl.when(wid == NW - 1)
        def _():
            pltpu.sync_copy(ttail_hbm, obuf.at[0, pl.ds(0, 16), :])
            pltpu.sync_copy(obuf.at[0, pl.ds(0, 16), :],
                            tlin_hbm.at[pl.ds(_NBLK * 32, 16), :])

    return tk


def _make_kernel(NC: int, NS: int):
    NW = NC * NS
    assert _B // NW == _BLK

    mesh = plsc.VectorSubcoreMesh(core_axis_name="c", subcore_axis_name="s")

    @functools.partial(
        pl.kernel,
        mesh=mesh,
        compiler_params=pltpu.CompilerParams(use_tc_tiling_on_sc=False,
                                             needs_layout_passes=False),
        out_type=jax.ShapeDtypeStruct((_T, _D // 8, _B // _BLK, 8, _BLK),
                                      jnp.float32),
        scratch_types=[
            pltpu.VMEM((3, _BLK, _TT), jnp.int32),
            pltpu.VMEM((3, _TT * _BLK), jnp.int32),
            pltpu.VMEM((3, _TT * _BLK, _D), jnp.float32),
            pltpu.VMEM((4, _D // 8, 8, _BLK + 1), jnp.float32),
            pltpu.SemaphoreType.DMA((3,)),
            pltpu.SemaphoreType.DMA((4,)),
        ],
    )
    def k(x_hbm, table_hbm, out_hbm, idxc, sl, gbuf, obuf, s_g, s_o):
        wid = lax.axis_index("s") * NC + lax.axis_index("c")
        lane = lax.iota(jnp.int32, 16)

        def prep_chunk(c, b):
            pltpu.sync_copy(
                x_hbm.at[pl.ds(wid * _BLK, _BLK), pl.ds(c * _TT, _TT)],
                idxc.at[b])

            @plsc.parallel_loop(0, _TT * 8, unroll=8)
            def _(kk):
                bi = lane + (kk & 7) * 16
                tv = jnp.zeros((16,), jnp.int32) + (kk >> 3)
                v = plsc.load_gather(idxc.at[b], [bi, tv])
                sl[b, pl.ds(kk * 16, 16)] = v

        def start_gather(b):
            return pltpu.async_copy(
                table_hbm.at[sl.at[b]], gbuf.at[b], s_g.at[b])

        def wait_gather(b):
            pltpu.make_async_copy(
                table_hbm.at[sl.at[b]], gbuf.at[b], s_g.at[b]).wait()

        g0 = lane >> 3
        di0 = lane & 7
        g1 = g0 + 2

        def repack_store(c, b):
            for tt in range(_TT):
                t = c * _TT + tt
                p = tt % 4

                @pl.when(c * _TT + tt >= 4)
                def _():
                    pltpu.make_async_copy(
                        obuf.at[p, :, :, pl.ds(0, _BLK)],
                        out_hbm.at[t, :, wid, :, :], s_o.at[p]).wait()

                ob = obuf.at[p]

                @plsc.parallel_loop(0, _BLK, unroll=16)
                def _(bi):
                    row = tt * _BLK + bi
                    bv = jnp.zeros((16,), jnp.int32) + bi
                    v0 = gbuf[b, row, pl.ds(0, 16)]
                    v1 = gbuf[b, row, pl.ds(16, 16)]
                    plsc.store_scatter(ob, [g0, di0, bv], v0)
                    plsc.store_scatter(ob, [g1, di0, bv], v1)

                pltpu.async_copy(
                    obuf.at[p, :, :, pl.ds(0, _BLK)],
                    out_hbm.at[t, :, wid, :, :], s_o.at[p])

        prep_chunk(0, 0)
        start_gather(0)
        prep_chunk(1, 1)
        start_gather(1)

        def body(c, carry):
            b = c % 3
            b2 = (c + 2) % 3
            wait_gather(b)

            @pl.when(c + 2 < _NCH)
            def _():
                prep_chunk(c + 2, b2)
                start_gather(b2)

            repack_store(c, b)
            return carry

        lax.fori_loop(0, _NCH, body, 0)
        for t in (_T - 4, _T - 3, _T - 2, _T - 1):
            pltpu.make_async_copy(
                obuf.at[t % 4, :, :, pl.ds(0, _BLK)],
                out_hbm.at[t, :, wid, :, :], s_o.at[t % 4]).wait()

    return k


def kernel(x, table):
    info = plsc.get_sparse_core_info()
    sk = _make_shuffle(info.num_cores, info.num_subcores)
    tk = _make_untile(info.num_cores, info.num_subcores)
    k = _make_kernel(info.num_cores, info.num_subcores)
    tbig = sk(table.T)
    ttail = table[_NBLK * 128:].reshape(16, 128)
    tlin = tk(tbig, ttail).reshape(_V, _D)
    out5 = k(x, tlin)
    return out5.transpose((2, 4, 0, 1, 3)).reshape(_B, _T, _D)


# speedup vs baseline: 1.1385x; 1.1385x over previous
import functools

import jax
import jax.numpy as jnp
from jax import lax
from jax.experimental import pallas as pl
from jax.experimental.pallas import tpu as pltpu
from jax.experimental.pallas import tpu_sc as plsc

_B = 4096
_T = 200
_D = 32
_BLK = 128
_TT = 8
_NCH = _T // _TT


_V = 1000000
_NBLK = _V // 128
_VTAIL = _V - _NBLK * 128


def _make_shuffle(NC: int, NS: int):
    NW = NC * NS
    PER_W = _NBLK // NW + 1

    mesh = plsc.VectorSubcoreMesh(core_axis_name="c", subcore_axis_name="s")

    @functools.partial(
        pl.kernel,
        mesh=mesh,
        compiler_params=pltpu.CompilerParams(use_tc_tiling_on_sc=True,
                                             needs_layout_passes=False),
        out_type=jax.ShapeDtypeStruct((_NBLK + 1, 32, 128), jnp.float32),
        scratch_types=[
            pltpu.VMEM((4, 32, 128), jnp.float32),
            pltpu.SemaphoreType.DMA((4,)),
            pltpu.SemaphoreType.DMA((4,)),
        ],
    )
    def sk(tT_hbm, tbig_hbm, vbuf, s_i, s_o):
        wid = lax.axis_index("s") * NC + lax.axis_index("c")

        def blk_of(i):
            return wid + NW * i

        def in_cp(i, b):
            return pltpu.make_async_copy(
                tT_hbm.at[:, pl.ds(blk_of(i) * 128, 128)], vbuf.at[b],
                s_i.at[b])

        def out_cp(i, b):
            return pltpu.make_async_copy(
                vbuf.at[b], tbig_hbm.at[blk_of(i)], s_o.at[b])

        for j in range(2):
            @pl.when(blk_of(j) < _NBLK)
            def _():
                in_cp(j, j).start()

        def body(i, carry):
            b = i % 4

            @pl.when(jnp.logical_and(i >= 2, blk_of(i - 2) < _NBLK))
            def _():
                out_cp(i - 2, (i - 2) % 4).wait()

            @pl.when(blk_of(i + 2) < _NBLK)
            def _():
                in_cp(i + 2, (i + 2) % 4).start()

            @pl.when(blk_of(i) < _NBLK)
            def _():
                in_cp(i, b).wait()
                out_cp(i, b).start()

            return carry

        lax.fori_loop(0, PER_W, body, 0)
        for i in (PER_W - 2, PER_W - 1):
            @pl.when(blk_of(i) < _NBLK)
            def _():
                out_cp(i, i % 4).wait()

    return sk


def _make_untile(NC: int, NS: int):
    NW = NC * NS
    PER_W = _NBLK // NW + 1

    mesh = plsc.VectorSubcoreMesh(core_axis_name="c", subcore_axis_name="s")

    @functools.partial(
        pl.kernel,
        mesh=mesh,
        compiler_params=pltpu.CompilerParams(use_tc_tiling_on_sc=False,
                                             needs_layout_passes=False),
        out_type=jax.ShapeDtypeStruct((_V // 4, 128), jnp.float32),
        scratch_types=[
            pltpu.VMEM((3, 32, 129), jnp.float32),
            pltpu.VMEM((2, 32, 128), jnp.float32),
            pltpu.SemaphoreType.DMA((3,)),
            pltpu.SemaphoreType.DMA((2,)),
        ],
    )
    def tk(tbig_hbm, ttail_hbm, tlin_hbm, sbuf, obuf, s_i, s_o):
        wid = lax.axis_index("s") * NC + lax.axis_index("c")
        lane = lax.iota(jnp.int32, 16)

        def blk_of(i):
            return wid + NW * i

        def in_cp(i, b):
            return pltpu.make_async_copy(
                tbig_hbm.at[blk_of(i)], sbuf.at[b, :, pl.ds(0, 128)],
                s_i.at[b])

        def out_cp(i, b):
            return pltpu.make_async_copy(
                obuf.at[b], tlin_hbm.at[pl.ds(blk_of(i) * 32, 32), :],
                s_o.at[b])

        def repack(b_in, b_out):
            sb = sbuf.at[b_in]

            @plsc.parallel_loop(0, 32 * 8, unroll=8)
            def _(kk):
                r = kk >> 3
                h = kk & 7
                rows = lane + (h & 1) * 16
                col = jnp.zeros((16,), jnp.int32) + (4 * r + (h >> 1))
                v = plsc.load_gather(sb, [rows, col])
                obuf[b_out, r, pl.ds(h * 16, 16)] = v

        for j in range(2):
            @pl.when(blk_of(j) < _NBLK)
            def _():
                in_cp(j, j).start()

        def body(i, carry):
            b = i % 3

            @pl.when(blk_of(i + 2) < _NBLK)
            def _():
                in_cp(i + 2, (i + 2) % 3).start()

            @pl.when(blk_of(i) < _NBLK)
            def _():
                in_cp(i, b).wait()

                @pl.when(i >= 2)
                def _():
                    out_cp(i - 2, i % 2).wait()

                repack(b, i % 2)
                out_cp(i, i % 2).start()

            return carry

        lax.fori_loop(0, PER_W, body, 0)
        for i in (PER_W - 2, PER_W - 1):
            @pl.when(blk_of(i) < _NBLK)
            def _():
                out_cp(i, i % 2).wait()

        @pl.when(wid == NW - 1)
        def _():
            pltpu.sync_copy(ttail_hbm, obuf.at[0, pl.ds(0, 16), :])
            pltpu.sync_copy(obuf.at[0, pl.ds(0, 16), :],
                            tlin_hbm.at[pl.ds(_NBLK * 32, 16), :])

    return tk


def _make_kernel(NC: int, NS: int):
    NW = NC * NS
    assert _B // NW == _BLK

    mesh = plsc.VectorSubcoreMesh(core_axis_name="c", subcore_axis_name="s")

    @functools.partial(
        pl.kernel,
        mesh=mesh,
        compiler_params=pltpu.CompilerParams(use_tc_tiling_on_sc=False,
                                             needs_layout_passes=False),
        out_type=jax.ShapeDtypeStruct((_T, _D // 8, _B // _BLK, 8, _BLK),
                                      jnp.float32),
        scratch_types=[
            pltpu.VMEM((3, _BLK, _TT), jnp.int32),
            pltpu.VMEM((3, _TT * _BLK), jnp.int32),
            pltpu.VMEM((3, _TT * _BLK, _D), jnp.float32),
            pltpu.VMEM((4, _D // 8, 8, _BLK + 1), jnp.float32),
            pltpu.SemaphoreType.DMA((3,)),
            pltpu.SemaphoreType.DMA((4,)),
        ],
    )
    def k(x_hbm, table_hbm, out_hbm, idxc, sl, gbuf, obuf, s_g, s_o):
        wid = lax.axis_index("s") * NC + lax.axis_index("c")
        lane = lax.iota(jnp.int32, 16)

        def prep_chunk(c, b):
            pltpu.sync_copy(
                x_hbm.at[pl.ds(wid * _BLK, _BLK), pl.ds(c * _TT, _TT)],
                idxc.at[b])

            @plsc.parallel_loop(0, _TT * 8, unroll=8)
            def _(kk):
                bi = lane + (kk & 7) * 16
                tv = jnp.zeros((16,), jnp.int32) + (kk >> 3)
                v = plsc.load_gather(idxc.at[b], [bi, tv])
                sl[b, pl.ds(kk * 16, 16)] = v

        def start_gather(b):
            return pltpu.async_copy(
                table_hbm.at[sl.at[b]], gbuf.at[b], s_g.at[b])

        def wait_gather(b):
            pltpu.make_async_copy(
                table_hbm.at[sl.at[b]], gbuf.at[b], s_g.at[b]).wait()

        g0 = lane >> 3
        di0 = lane & 7
        g1 = g0 + 2

        def repack_store(c, b):
            for tt in range(_TT):
                t = c * _TT + tt
                p = tt % 4

                @pl.when(c * _TT + tt >= 4)
                def _():
                    pltpu.make_async_copy(
                        obuf.at[p, :, :, pl.ds(0, _BLK)],
                        out_hbm.at[t, :, wid, :, :], s_o.at[p]).wait()

                ob = obuf.at[p]

                @plsc.parallel_loop(0, _BLK, unroll=8)
                def _(bi):
                    row = tt * _BLK + bi
                    bv = jnp.zeros((16,), jnp.int32) + bi
                    v0 = gbuf[b, row, pl.ds(0, 16)]
                    v1 = gbuf[b, row, pl.ds(16, 16)]
                    plsc.store_scatter(ob, [g0, di0, bv], v0)
                    plsc.store_scatter(ob, [g1, di0, bv], v1)

                pltpu.async_copy(
                    obuf.at[p, :, :, pl.ds(0, _BLK)],
                    out_hbm.at[t, :, wid, :, :], s_o.at[p])

        prep_chunk(0, 0)
        start_gather(0)
        prep_chunk(1, 1)
        start_gather(1)

        def body(c, carry):
            b = c % 3
            b2 = (c + 2) % 3
            wait_gather(b)

            @pl.when(c + 2 < _NCH)
            def _():
                prep_chunk(c + 2, b2)
                start_gather(b2)

            repack_store(c, b)
            return carry

        lax.fori_loop(0, _NCH, body, 0)
        for t in (_T - 4, _T - 3, _T - 2, _T - 1):
            pltpu.make_async_copy(
                obuf.at[t % 4, :, :, pl.ds(0, _BLK)],
                out_hbm.at[t, :, wid, :, :], s_o.at[t % 4]).wait()

    return k


def kernel(x, table):
    info = plsc.get_sparse_core_info()
    sk = _make_shuffle(info.num_cores, info.num_subcores)
    tk = _make_untile(info.num_cores, info.num_subcores)
    k = _make_kernel(info.num_cores, info.num_subcores)
    tbig = sk(table.T)
    ttail = table[_NBLK * 128:].reshape(16, 128)
    tlin = tk(tbig, ttail).reshape(_V, _D)
    out5 = k(x, tlin)
    return out5.transpose((2, 4, 0, 1, 3)).reshape(_B, _T, _D)


# speedup vs baseline: 1.1821x; 1.0384x over previous
import functools

import jax
import jax.numpy as jnp
from jax import lax
from jax.experimental import pallas as pl
from jax.experimental.pallas import tpu as pltpu
from jax.experimental.pallas import tpu_sc as plsc

_B = 4096
_T = 200
_D = 32
_BLK = 128
_TT = 8
_NCH = _T // _TT


_V = 1000000
_NBLK = _V // 128
_VTAIL = _V - _NBLK * 128


def _make_shuffle(NC: int, NS: int):
    NW = NC * NS
    PER_W = _NBLK // NW + 1

    mesh = plsc.VectorSubcoreMesh(core_axis_name="c", subcore_axis_name="s")

    @functools.partial(
        pl.kernel,
        mesh=mesh,
        compiler_params=pltpu.CompilerParams(use_tc_tiling_on_sc=True,
                                             needs_layout_passes=False),
        out_type=jax.ShapeDtypeStruct((_NBLK + 1, 32, 128), jnp.float32),
        scratch_types=[
            pltpu.VMEM((6, 32, 128), jnp.float32),
            pltpu.SemaphoreType.DMA((6,)),
            pltpu.SemaphoreType.DMA((6,)),
        ],
    )
    def sk(tT_hbm, tbig_hbm, vbuf, s_i, s_o):
        wid = lax.axis_index("s") * NC + lax.axis_index("c")

        def blk_of(i):
            return wid + NW * i

        def in_cp(i, b):
            return pltpu.make_async_copy(
                tT_hbm.at[:, pl.ds(blk_of(i) * 128, 128)], vbuf.at[b],
                s_i.at[b])

        def out_cp(i, b):
            return pltpu.make_async_copy(
                vbuf.at[b], tbig_hbm.at[blk_of(i)], s_o.at[b])

        for j in range(3):
            @pl.when(blk_of(j) < _NBLK)
            def _():
                in_cp(j, j).start()

        def body(i, carry):
            b = i % 6

            @pl.when(jnp.logical_and(i >= 3, blk_of(i - 3) < _NBLK))
            def _():
                out_cp(i - 3, (i - 3) % 6).wait()

            @pl.when(blk_of(i + 3) < _NBLK)
            def _():
                in_cp(i + 3, (i + 3) % 6).start()

            @pl.when(blk_of(i) < _NBLK)
            def _():
                in_cp(i, b).wait()
                out_cp(i, b).start()

            return carry

        lax.fori_loop(0, PER_W, body, 0)
        for i in (PER_W - 3, PER_W - 2, PER_W - 1):
            @pl.when(blk_of(i) < _NBLK)
            def _():
                out_cp(i, i % 6).wait()

    return sk


def _make_untile(NC: int, NS: int):
    NW = NC * NS
    PER_W = _NBLK // NW + 1

    mesh = plsc.VectorSubcoreMesh(core_axis_name="c", subcore_axis_name="s")

    @functools.partial(
        pl.kernel,
        mesh=mesh,
        compiler_params=pltpu.CompilerParams(use_tc_tiling_on_sc=False,
                                             needs_layout_passes=False),
        out_type=jax.ShapeDtypeStruct((_V // 4, 128), jnp.float32),
        scratch_types=[
            pltpu.VMEM((3, 32, 129), jnp.float32),
            pltpu.VMEM((2, 32, 128), jnp.float32),
            pltpu.SemaphoreType.DMA((3,)),
            pltpu.SemaphoreType.DMA((2,)),
        ],
    )
    def tk(tbig_hbm, ttail_hbm, tlin_hbm, sbuf, obuf, s_i, s_o):
        wid = lax.axis_index("s") * NC + lax.axis_index("c")
        lane = lax.iota(jnp.int32, 16)

        def blk_of(i):
            return wid + NW * i

        def in_cp(i, b):
            return pltpu.make_async_copy(
                tbig_hbm.at[blk_of(i)], sbuf.at[b, :, pl.ds(0, 128)],
                s_i.at[b])

        def out_cp(i, b):
            return pltpu.make_async_copy(
                obuf.at[b], tlin_hbm.at[pl.ds(blk_of(i) * 32, 32), :],
                s_o.at[b])

        def repack(b_in, b_out):
            sb = sbuf.at[b_in]

            @plsc.parallel_loop(0, 32 * 8, unroll=8)
            def _(kk):
                r = kk >> 3
                h = kk & 7
                rows = lane + (h & 1) * 16
                col = jnp.zeros((16,), jnp.int32) + (4 * r + (h >> 1))
                v = plsc.load_gather(sb, [rows, col])
                obuf[b_out, r, pl.ds(h * 16, 16)] = v

        for j in range(2):
            @pl.when(blk_of(j) < _NBLK)
            def _():
                in_cp(j, j).start()

        def body(i, carry):
            b = i % 3

            @pl.when(blk_of(i + 2) < _NBLK)
            def _():
                in_cp(i + 2, (i + 2) % 3).start()

            @pl.when(blk_of(i) < _NBLK)
            def _():
                in_cp(i, b).wait()

                @pl.when(i >= 2)
                def _():
                    out_cp(i - 2, i % 2).wait()

                repack(b, i % 2)
                out_cp(i, i % 2).start()

            return carry

        lax.fori_loop(0, PER_W, body, 0)
        for i in (PER_W - 2, PER_W - 1):
            @pl.when(blk_of(i) < _NBLK)
            def _():
                out_cp(i, i % 2).wait()

        @pl.when(wid == NW - 1)
        def _():
            pltpu.sync_copy(ttail_hbm, obuf.at[0, pl.ds(0, 16), :])
            pltpu.sync_copy(obuf.at[0, pl.ds(0, 16), :],
                            tlin_hbm.at[pl.ds(_NBLK * 32, 16), :])

    return tk


def _make_kernel(NC: int, NS: int):
    NW = NC * NS
    assert _B // NW == _BLK

    mesh = plsc.VectorSubcoreMesh(core_axis_name="c", subcore_axis_name="s")

    @functools.partial(
        pl.kernel,
        mesh=mesh,
        compiler_params=pltpu.CompilerParams(use_tc_tiling_on_sc=False,
                                             needs_layout_passes=False),
        out_type=jax.ShapeDtypeStruct((_T, _D // 8, _B // _BLK, 8, _BLK),
                                      jnp.float32),
        scratch_types=[
            pltpu.VMEM((3, _BLK, _TT), jnp.int32),
            pltpu.VMEM((3, _TT * _BLK), jnp.int32),
            pltpu.VMEM((3, _TT * _BLK, _D), jnp.float32),
            pltpu.VMEM((4, _D // 8, 8, _BLK + 1), jnp.float32),
            pltpu.SemaphoreType.DMA((3,)),
            pltpu.SemaphoreType.DMA((4,)),
        ],
    )
    def k(x_hbm, table_hbm, out_hbm, idxc, sl, gbuf, obuf, s_g, s_o):
        wid = lax.axis_index("s") * NC + lax.axis_index("c")
        lane = lax.iota(jnp.int32, 16)

        def prep_chunk(c, b):
            pltpu.sync_copy(
                x_hbm.at[pl.ds(wid * _BLK, _BLK), pl.ds(c * _TT, _TT)],
                idxc.at[b])

            @plsc.parallel_loop(0, _TT * 8, unroll=8)
            def _(kk):
                bi = lane + (kk & 7) * 16
                tv = jnp.zeros((16,), jnp.int32) + (kk >> 3)
                v = plsc.load_gather(idxc.at[b], [bi, tv])
                sl[b, pl.ds(kk * 16, 16)] = v

        def start_gather(b):
            return pltpu.async_copy(
                table_hbm.at[sl.at[b]], gbuf.at[b], s_g.at[b])

        def wait_gather(b):
            pltpu.make_async_copy(
                table_hbm.at[sl.at[b]], gbuf.at[b], s_g.at[b]).wait()

        g0 = lane >> 3
        di0 = lane & 7
        g1 = g0 + 2

        def repack_store(c, b):
            for tt in range(_TT):
                t = c * _TT + tt
                p = tt % 4

                @pl.when(c * _TT + tt >= 4)
                def _():
                    pltpu.make_async_copy(
                        obuf.at[p, :, :, pl.ds(0, _BLK)],
                        out_hbm.at[t, :, wid, :, :], s_o.at[p]).wait()

                ob = obuf.at[p]

                @plsc.parallel_loop(0, _BLK, unroll=8)
                def _(bi):
                    row = tt * _BLK + bi
                    bv = jnp.zeros((16,), jnp.int32) + bi
                    v0 = gbuf[b, row, pl.ds(0, 16)]
                    v1 = gbuf[b, row, pl.ds(16, 16)]
                    plsc.store_scatter(ob, [g0, di0, bv], v0)
                    plsc.store_scatter(ob, [g1, di0, bv], v1)

                pltpu.async_copy(
                    obuf.at[p, :, :, pl.ds(0, _BLK)],
                    out_hbm.at[t, :, wid, :, :], s_o.at[p])

        prep_chunk(0, 0)
        start_gather(0)
        prep_chunk(1, 1)
        start_gather(1)

        def body(c, carry):
            b = c % 3
            b2 = (c + 2) % 3
            wait_gather(b)

            @pl.when(c + 2 < _NCH)
            def _():
                prep_chunk(c + 2, b2)
                start_gather(b2)

            repack_store(c, b)
            return carry

        lax.fori_loop(0, _NCH, body, 0)
        for t in (_T - 4, _T - 3, _T - 2, _T - 1):
            pltpu.make_async_copy(
                obuf.at[t % 4, :, :, pl.ds(0, _BLK)],
                out_hbm.at[t, :, wid, :, :], s_o.at[t % 4]).wait()

    return k


def kernel(x, table):
    info = plsc.get_sparse_core_info()
    sk = _make_shuffle(info.num_cores, info.num_subcores)
    tk = _make_untile(info.num_cores, info.num_subcores)
    k = _make_kernel(info.num_cores, info.num_subcores)
    tbig = sk(table.T)
    ttail = table[_NBLK * 128:].reshape(16, 128)
    tlin = tk(tbig, ttail).reshape(_V, _D)
    out5 = k(x, tlin)
    return out5.transpose((2, 4, 0, 1, 3)).reshape(_B, _T, _D)


# speedup vs baseline: 1.1847x; 1.0022x over previous
import functools

import jax
import jax.numpy as jnp
from jax import lax
from jax.experimental import pallas as pl
from jax.experimental.pallas import tpu as pltpu
from jax.experimental.pallas import tpu_sc as plsc

_B = 4096
_T = 200
_D = 32
_BLK = 128
_TT = 8
_NCH = _T // _TT


_V = 1000000
_NBLK = _V // 128
_VTAIL = _V - _NBLK * 128


def _make_shuffle(NC: int, NS: int):
    NW = NC * NS
    PER_W = _NBLK // NW + 1

    mesh = plsc.VectorSubcoreMesh(core_axis_name="c", subcore_axis_name="s")

    @functools.partial(
        pl.kernel,
        mesh=mesh,
        compiler_params=pltpu.CompilerParams(use_tc_tiling_on_sc=True,
                                             needs_layout_passes=False),
        out_type=jax.ShapeDtypeStruct((_NBLK + 1, 32, 128), jnp.float32),
        scratch_types=[
            pltpu.VMEM((6, 32, 128), jnp.float32),
            pltpu.SemaphoreType.DMA((6,)),
            pltpu.SemaphoreType.DMA((6,)),
        ],
    )
    def sk(tT_hbm, tbig_hbm, vbuf, s_i, s_o):
        wid = lax.axis_index("s") * NC + lax.axis_index("c")

        def blk_of(i):
            return wid + NW * i

        def in_cp(i, b):
            return pltpu.make_async_copy(
                tT_hbm.at[:, pl.ds(blk_of(i) * 128, 128)], vbuf.at[b],
                s_i.at[b])

        def out_cp(i, b):
            return pltpu.make_async_copy(
                vbuf.at[b], tbig_hbm.at[blk_of(i)], s_o.at[b])

        for j in range(3):
            @pl.when(blk_of(j) < _NBLK)
            def _():
                in_cp(j, j).start()

        def body(i, carry):
            b = i % 6

            @pl.when(jnp.logical_and(i >= 3, blk_of(i - 3) < _NBLK))
            def _():
                out_cp(i - 3, (i - 3) % 6).wait()

            @pl.when(blk_of(i + 3) < _NBLK)
            def _():
                in_cp(i + 3, (i + 3) % 6).start()

            @pl.when(blk_of(i) < _NBLK)
            def _():
                in_cp(i, b).wait()
                out_cp(i, b).start()

            return carry

        lax.fori_loop(0, PER_W, body, 0)
        for i in (PER_W - 3, PER_W - 2, PER_W - 1):
            @pl.when(blk_of(i) < _NBLK)
            def _():
                out_cp(i, i % 6).wait()

    return sk


def _make_untile(NC: int, NS: int):
    NW = NC * NS
    PER_W = _NBLK // NW + 1

    mesh = plsc.VectorSubcoreMesh(core_axis_name="c", subcore_axis_name="s")

    @functools.partial(
        pl.kernel,
        mesh=mesh,
        compiler_params=pltpu.CompilerParams(use_tc_tiling_on_sc=False,
                                             needs_layout_passes=False),
        out_type=jax.ShapeDtypeStruct((_V // 4, 128), jnp.float32),
        scratch_types=[
            pltpu.VMEM((3, 32, 129), jnp.float32),
            pltpu.VMEM((3, 32, 128), jnp.float32),
            pltpu.SemaphoreType.DMA((3,)),
            pltpu.SemaphoreType.DMA((3,)),
        ],
    )
    def tk(tbig_hbm, ttail_hbm, tlin_hbm, sbuf, obuf, s_i, s_o):
        wid = lax.axis_index("s") * NC + lax.axis_index("c")
        lane = lax.iota(jnp.int32, 16)

        def blk_of(i):
            return wid + NW * i

        def in_cp(i, b):
            return pltpu.make_async_copy(
                tbig_hbm.at[blk_of(i)], sbuf.at[b, :, pl.ds(0, 128)],
                s_i.at[b])

        def out_cp(i, b):
            return pltpu.make_async_copy(
                obuf.at[b], tlin_hbm.at[pl.ds(blk_of(i) * 32, 32), :],
                s_o.at[b])

        def repack(b_in, b_out):
            sb = sbuf.at[b_in]

            @plsc.parallel_loop(0, 32 * 8, unroll=8)
            def _(kk):
                r = kk >> 3
                h = kk & 7
                rows = lane + (h & 1) * 16
                col = jnp.zeros((16,), jnp.int32) + (4 * r + (h >> 1))
                v = plsc.load_gather(sb, [rows, col])
                obuf[b_out, r, pl.ds(h * 16, 16)] = v

        for j in range(2):
            @pl.when(blk_of(j) < _NBLK)
            def _():
                in_cp(j, j).start()

        def body(i, carry):
            b = i % 3

            @pl.when(blk_of(i + 2) < _NBLK)
            def _():
                in_cp(i + 2, (i + 2) % 3).start()

            @pl.when(blk_of(i) < _NBLK)
            def _():
                in_cp(i, b).wait()

                @pl.when(i >= 3)
                def _():
                    out_cp(i - 3, i % 3).wait()

                repack(b, i % 3)
                out_cp(i, i % 3).start()

            return carry

        lax.fori_loop(0, PER_W, body, 0)
        for i in (PER_W - 4, PER_W - 3, PER_W - 2, PER_W - 1):
            @pl.when(jnp.logical_and(blk_of(i) < _NBLK,
                                     blk_of(i + 3) >= _NBLK))
            def _():
                out_cp(i, i % 3).wait()

        @pl.when(wid == NW - 1)
        def _():
            pltpu.sync_copy(ttail_hbm, obuf.at[0, pl.ds(0, 16), :])
            pltpu.sync_copy(obuf.at[0, pl.ds(0, 16), :],
                            tlin_hbm.at[pl.ds(_NBLK * 32, 16), :])

    return tk


def _make_kernel(NC: int, NS: int):
    NW = NC * NS
    assert _B // NW == _BLK

    mesh = plsc.VectorSubcoreMesh(core_axis_name="c", subcore_axis_name="s")

    @functools.partial(
        pl.kernel,
        mesh=mesh,
        compiler_params=pltpu.CompilerParams(use_tc_tiling_on_sc=False,
                                             needs_layout_passes=False),
        out_type=jax.ShapeDtypeStruct((_T, _D // 8, _B // _BLK, 8, _BLK),
                                      jnp.float32),
        scratch_types=[
            pltpu.VMEM((3, _BLK, _TT), jnp.int32),
            pltpu.VMEM((3, _TT * _BLK), jnp.int32),
            pltpu.VMEM((3, _TT * _BLK, _D), jnp.float32),
            pltpu.VMEM((4, _D // 8, 8, _BLK + 1), jnp.float32),
            pltpu.SemaphoreType.DMA((3,)),
            pltpu.SemaphoreType.DMA((4,)),
        ],
    )
    def k(x_hbm, table_hbm, out_hbm, idxc, sl, gbuf, obuf, s_g, s_o):
        wid = lax.axis_index("s") * NC + lax.axis_index("c")
        lane = lax.iota(jnp.int32, 16)

        def prep_chunk(c, b):
            pltpu.sync_copy(
                x_hbm.at[pl.ds(wid * _BLK, _BLK), pl.ds(c * _TT, _TT)],
                idxc.at[b])

            @plsc.parallel_loop(0, _TT * 8, unroll=8)
            def _(kk):
                bi = lane + (kk & 7) * 16
                tv = jnp.zeros((16,), jnp.int32) + (kk >> 3)
                v = plsc.load_gather(idxc.at[b], [bi, tv])
                sl[b, pl.ds(kk * 16, 16)] = v

        def start_gather(b):
            return pltpu.async_copy(
                table_hbm.at[sl.at[b]], gbuf.at[b], s_g.at[b])

        def wait_gather(b):
            pltpu.make_async_copy(
                table_hbm.at[sl.at[b]], gbuf.at[b], s_g.at[b]).wait()

        g0 = lane >> 3
        di0 = lane & 7
        g1 = g0 + 2

        def repack_store(c, b):
            for tt in range(_TT):
                t = c * _TT + tt
                p = tt % 4

                @pl.when(c * _TT + tt >= 4)
                def _():
                    pltpu.make_async_copy(
                        obuf.at[p, :, :, pl.ds(0, _BLK)],
                        out_hbm.at[t, :, wid, :, :], s_o.at[p]).wait()

                ob = obuf.at[p]

                @plsc.parallel_loop(0, _BLK, unroll=8)
                def _(bi):
                    row = tt * _BLK + bi
                    bv = jnp.zeros((16,), jnp.int32) + bi
                    v0 = gbuf[b, row, pl.ds(0, 16)]
                    v1 = gbuf[b, row, pl.ds(16, 16)]
                    plsc.store_scatter(ob, [g0, di0, bv], v0)
                    plsc.store_scatter(ob, [g1, di0, bv], v1)

                pltpu.async_copy(
                    obuf.at[p, :, :, pl.ds(0, _BLK)],
                    out_hbm.at[t, :, wid, :, :], s_o.at[p])

        prep_chunk(0, 0)
        start_gather(0)
        prep_chunk(1, 1)
        start_gather(1)

        def body(c, carry):
            b = c % 3
            b2 = (c + 2) % 3
            wait_gather(b)

            @pl.when(c + 2 < _NCH)
            def _():
                prep_chunk(c + 2, b2)
                start_gather(b2)

            repack_store(c, b)
            return carry

        lax.fori_loop(0, _NCH, body, 0)
        for t in (_T - 4, _T - 3, _T - 2, _T - 1):
            pltpu.make_async_copy(
                obuf.at[t % 4, :, :, pl.ds(0, _BLK)],
                out_hbm.at[t, :, wid, :, :], s_o.at[t % 4]).wait()

    return k


def kernel(x, table):
    info = plsc.get_sparse_core_info()
    sk = _make_shuffle(info.num_cores, info.num_subcores)
    tk = _make_untile(info.num_cores, info.num_subcores)
    k = _make_kernel(info.num_cores, info.num_subcores)
    tbig = sk(table.T)
    ttail = table[_NBLK * 128:].reshape(16, 128)
    tlin = tk(tbig, ttail).reshape(_V, _D)
    out5 = k(x, tlin)
    return out5.transpose((2, 4, 0, 1, 3)).reshape(_B, _T, _D)
